# baseline (speedup 1.0000x reference)
import jax
import jax.numpy as jnp
from jax.experimental import pallas as pl
from jax.experimental.pallas import tpu as pltpu

_TOPK = 100
_C = 91
_NCHUNK = 1024
_CHUNK = 80


def _topk_body(pa_ref, pb_ref, boxes_ref, ts_ref, scores_ref, labels_ref,
               boxes_out_ref, pwork_ref):
    pwork_ref[...] = pb_ref[0]
    pa = pa_ref[0]
    cm0 = jnp.max(pa, axis=0, keepdims=True)
    jio = jax.lax.broadcasted_iota(jnp.int32, (_CHUNK, _NCHUNK), 0)
    ai0 = jnp.min(jnp.where(pa == cm0, jio, _CHUNK), axis=0, keepdims=True)
    cio = jax.lax.broadcasted_iota(jnp.int32, (1, _NCHUNK), 1)
    lio = jax.lax.broadcasted_iota(jnp.int32, (1, _CHUNK), 1)

    tsf = ts_ref[...].reshape(1, 2).astype(jnp.float32)
    wh = jnp.concatenate([tsf[:, 1:2], tsf[:, 0:1]], axis=1)
    scale = jnp.concatenate([wh, wh], axis=1)

    def step(i, carry):
        cm, ai = carry
        m = jnp.max(cm)
        ci = jnp.min(jnp.where(cm == m, cio, _NCHUNK))
        j = jnp.min(jnp.where(cio == ci, ai, _CHUNK))
        flat = ci * _CHUNK + j
        q = flat // _C
        cls = flat % _C
        scores_ref[:, pl.ds(i, 1), :] = jnp.full((1, 1, 1), m, jnp.float32)
        labels_ref[:, pl.ds(i, 1), :] = jnp.full((1, 1, 1), cls, jnp.int32)
        b = boxes_ref[slice(0, 1), pl.ds(q, 1), :].reshape(1, 4)
        cxy = b[:, 0:2]
        bwh = b[:, 2:4]
        xyxy = jnp.concatenate([cxy - 0.5 * bwh, cxy + 0.5 * bwh], axis=1)
        xyxy = (xyxy * scale).astype(jnp.int32)
        boxes_out_ref[:, pl.ds(i, 1), :] = xyxy.reshape(1, 1, 4)
        row = pwork_ref[pl.ds(ci, 1), :]
        row = jnp.where(lio == j, -1.0, row)
        pwork_ref[pl.ds(ci, 1), :] = row
        nm = jnp.max(row)
        na = jnp.min(jnp.where(row == nm, lio, _CHUNK))
        cm = jnp.where(cio == ci, nm, cm)
        ai = jnp.where(cio == ci, na, ai)
        return cm, ai

    jax.lax.fori_loop(0, _TOPK, step, (cm0, ai0))


@jax.jit
def kernel(pred_logits, pred_boxes, target_sizes):
    B, Q, C = pred_logits.shape
    prob = jax.nn.sigmoid(pred_logits).reshape(B, Q * C)
    flat = jnp.pad(prob, ((0, 0), (0, _NCHUNK * _CHUNK - Q * C)),
                   constant_values=-1.0)
    pb = flat.reshape(B, _NCHUNK, _CHUNK)
    pa = pb.transpose(0, 2, 1)
    scores, labels, boxes = pl.pallas_call(
        _topk_body,
        grid=(B,),
        in_specs=[
            pl.BlockSpec((1, _CHUNK, _NCHUNK), lambda b: (b, 0, 0)),
            pl.BlockSpec((1, _NCHUNK, _CHUNK), lambda b: (b, 0, 0)),
            pl.BlockSpec((1, Q, 4), lambda b: (b, 0, 0)),
            pl.BlockSpec((1, 1, 2), lambda b: (b, 0, 0)),
        ],
        out_specs=[
            pl.BlockSpec((1, _TOPK, 1), lambda b: (b, 0, 0)),
            pl.BlockSpec((1, _TOPK, 1), lambda b: (b, 0, 0)),
            pl.BlockSpec((1, _TOPK, 4), lambda b: (b, 0, 0)),
        ],
        out_shape=[
            jax.ShapeDtypeStruct((B, _TOPK, 1), jnp.float32),
            jax.ShapeDtypeStruct((B, _TOPK, 1), jnp.int32),
            jax.ShapeDtypeStruct((B, _TOPK, 4), jnp.int32),
        ],
        scratch_shapes=[pltpu.VMEM((_NCHUNK, _CHUNK), jnp.float32)],
    )(pa, pb, pred_boxes, target_sizes.reshape(B, 1, 2))
    return scores[..., 0], labels[..., 0], boxes


# speedup vs baseline: 1.1283x; 1.1283x over previous
import jax
import jax.numpy as jnp
from jax.experimental import pallas as pl
from jax.experimental.pallas import tpu as pltpu

_TOPK = 100
_C = 91
_NCHUNK = 1024
_CHUNK = 80
_IMG_PER_BLK = 8


def _topk_body(pa_ref, pb_ref, boxes_ref, ts_ref, scores_ref, labels_ref,
               boxes_out_ref, pwork_ref):
    pwork_ref[...] = pb_ref[...]
    jio = jax.lax.broadcasted_iota(jnp.int32, (_CHUNK, _NCHUNK), 0)
    cio = jax.lax.broadcasted_iota(jnp.int32, (1, _NCHUNK), 1)
    lio = jax.lax.broadcasted_iota(jnp.int32, (1, _CHUNK), 1)

    cms, ais, scales = [], [], []
    for b in range(_IMG_PER_BLK):
        pa = pa_ref[b]
        cm0 = jnp.max(pa, axis=0, keepdims=True)
        ai0 = jnp.min(jnp.where(pa == cm0, jio, _CHUNK), axis=0,
                      keepdims=True)
        cms.append(cm0)
        ais.append(ai0)
        tsf = ts_ref[b].reshape(1, 2).astype(jnp.float32)
        wh = jnp.concatenate([tsf[:, 1:2], tsf[:, 0:1]], axis=1)
        scales.append(jnp.concatenate([wh, wh], axis=1))

    def step(i, carry):
        cms, ais = carry
        ncms, nais = [], []
        for b in range(_IMG_PER_BLK):
            cm, ai = cms[b], ais[b]
            m = jnp.max(cm)
            ci = jnp.min(jnp.where(cm == m, cio, _NCHUNK))
            j = jnp.min(jnp.where(cio == ci, ai, _CHUNK))
            flat = ci * _CHUNK + j
            q = flat // _C
            cls = flat % _C
            scores_ref[b, pl.ds(i, 1), :] = jnp.full((1, 1), m, jnp.float32)
            labels_ref[b, pl.ds(i, 1), :] = jnp.full((1, 1), cls, jnp.int32)
            bx = boxes_ref[b, pl.ds(q, 1), :]
            cxy = bx[:, 0:2]
            bwh = bx[:, 2:4]
            xyxy = jnp.concatenate([cxy - 0.5 * bwh, cxy + 0.5 * bwh], axis=1)
            boxes_out_ref[b, pl.ds(i, 1), :] = (
                xyxy * scales[b]).astype(jnp.int32)
            row = pwork_ref[b, pl.ds(ci, 1), :]
            row = jnp.where(lio == j, -1.0, row)
            pwork_ref[b, pl.ds(ci, 1), :] = row
            nm = jnp.max(row)
            na = jnp.min(jnp.where(row == nm, lio, _CHUNK))
            ncms.append(jnp.where(cio == ci, nm, cm))
            nais.append(jnp.where(cio == ci, na, ai))
        return ncms, nais

    jax.lax.fori_loop(0, _TOPK, step, (cms, ais))


@jax.jit
def kernel(pred_logits, pred_boxes, target_sizes):
    B, Q, C = pred_logits.shape
    prob = jax.nn.sigmoid(pred_logits).reshape(B, Q * C)
    flat = jnp.pad(prob, ((0, 0), (0, _NCHUNK * _CHUNK - Q * C)),
                   constant_values=-1.0)
    pb = flat.reshape(B, _NCHUNK, _CHUNK)
    pa = pb.transpose(0, 2, 1)
    g = _IMG_PER_BLK
    scores, labels, boxes = pl.pallas_call(
        _topk_body,
        grid=(B // g,),
        in_specs=[
            pl.BlockSpec((g, _CHUNK, _NCHUNK), lambda b: (b, 0, 0)),
            pl.BlockSpec((g, _NCHUNK, _CHUNK), lambda b: (b, 0, 0)),
            pl.BlockSpec((g, Q, 4), lambda b: (b, 0, 0)),
            pl.BlockSpec((g, 1, 2), lambda b: (b, 0, 0)),
        ],
        out_specs=[
            pl.BlockSpec((g, _TOPK, 1), lambda b: (b, 0, 0)),
            pl.BlockSpec((g, _TOPK, 1), lambda b: (b, 0, 0)),
            pl.BlockSpec((g, _TOPK, 4), lambda b: (b, 0, 0)),
        ],
        out_shape=[
            jax.ShapeDtypeStruct((B, _TOPK, 1), jnp.float32),
            jax.ShapeDtypeStruct((B, _TOPK, 1), jnp.int32),
            jax.ShapeDtypeStruct((B, _TOPK, 4), jnp.int32),
        ],
        scratch_shapes=[pltpu.VMEM((g, _NCHUNK, _CHUNK), jnp.float32)],
    )(pa, pb, pred_boxes, target_sizes.reshape(B, 1, 2))
    return scores[..., 0], labels[..., 0], boxes


# speedup vs baseline: 7.5511x; 6.6927x over previous
import jax
import jax.numpy as jnp
from jax.experimental import pallas as pl
from jax.experimental.pallas import tpu as pltpu

_TOPK = 100
_C = 91
_NCHUNK = 1024
_CHUNK = 80
_G = 8


def _topk_body(pa_ref, pb_ref, boxes_ref, ts_ref, scores_ref, labels_ref,
               boxes_out_ref, pwork_ref):
    pwork_ref[...] = pb_ref[...]
    jio = jax.lax.broadcasted_iota(jnp.int32, (_CHUNK, _NCHUNK), 0)
    cio = jax.lax.broadcasted_iota(jnp.int32, (1, _NCHUNK), 1)
    lio = jax.lax.broadcasted_iota(jnp.int32, (1, _CHUNK), 1)

    cm_rows, ai_rows = [], []
    for b in range(_G):
        pa = pa_ref[b]
        cm0 = jnp.max(pa, axis=0, keepdims=True)
        ai0 = jnp.min(jnp.where(pa == cm0, jio, _CHUNK), axis=0,
                      keepdims=True)
        cm_rows.append(cm0)
        ai_rows.append(ai0)
    cm_all = jnp.concatenate(cm_rows, axis=0)
    ai_all = jnp.concatenate(ai_rows, axis=0)

    tsf = ts_ref[...].reshape(_G, 2).astype(jnp.float32)
    wh = jnp.concatenate([tsf[:, 1:2], tsf[:, 0:1]], axis=1)
    scale = jnp.concatenate([wh, wh], axis=1)

    def step(i, carry):
        cm_all, ai_all = carry
        m_col = jnp.max(cm_all, axis=1, keepdims=True)
        ci_col = jnp.min(jnp.where(cm_all == m_col, cio, _NCHUNK),
                         axis=1, keepdims=True)
        j_col = jnp.min(jnp.where(cio == ci_col, ai_all, _CHUNK),
                        axis=1, keepdims=True)
        flat = ci_col * _CHUNK + j_col
        q_col = flat // _C
        cls_col = flat % _C
        scores_ref[:, pl.ds(i, 1), :] = m_col.reshape(_G, 1, 1)
        labels_ref[:, pl.ds(i, 1), :] = cls_col.reshape(_G, 1, 1)

        box_rows, work_rows = [], []
        for b in range(_G):
            box_rows.append(boxes_ref[b, pl.ds(q_col[b, 0], 1), :])
            work_rows.append(pwork_ref[b, pl.ds(ci_col[b, 0], 1), :])
        bx = jnp.concatenate(box_rows, axis=0)
        rows = jnp.concatenate(work_rows, axis=0)

        cxy = bx[:, 0:2]
        bwh = bx[:, 2:4]
        xyxy = jnp.concatenate([cxy - 0.5 * bwh, cxy + 0.5 * bwh], axis=1)
        xyxy = (xyxy * scale).astype(jnp.int32)

        rows = jnp.where(lio == j_col, -1.0, rows)
        nm_col = jnp.max(rows, axis=1, keepdims=True)
        na_col = jnp.min(jnp.where(rows == nm_col, lio, _CHUNK),
                         axis=1, keepdims=True)
        for b in range(_G):
            boxes_out_ref[b, pl.ds(i, 1), :] = xyxy[b:b + 1]
            pwork_ref[b, pl.ds(ci_col[b, 0], 1), :] = rows[b:b + 1]

        cm_all = jnp.where(cio == ci_col, nm_col, cm_all)
        ai_all = jnp.where(cio == ci_col, na_col, ai_all)
        return cm_all, ai_all

    jax.lax.fori_loop(0, _TOPK, step, (cm_all, ai_all))


@jax.jit
def kernel(pred_logits, pred_boxes, target_sizes):
    B, Q, C = pred_logits.shape
    prob = jax.nn.sigmoid(pred_logits).reshape(B, Q * C)
    flat = jnp.pad(prob, ((0, 0), (0, _NCHUNK * _CHUNK - Q * C)),
                   constant_values=-1.0)
    pb = flat.reshape(B, _NCHUNK, _CHUNK)
    pa = pb.transpose(0, 2, 1)
    scores, labels, boxes = pl.pallas_call(
        _topk_body,
        grid=(B // _G,),
        in_specs=[
            pl.BlockSpec((_G, _CHUNK, _NCHUNK), lambda b: (b, 0, 0)),
            pl.BlockSpec((_G, _NCHUNK, _CHUNK), lambda b: (b, 0, 0)),
            pl.BlockSpec((_G, Q, 4), lambda b: (b, 0, 0)),
            pl.BlockSpec((_G, 1, 2), lambda b: (b, 0, 0)),
        ],
        out_specs=[
            pl.BlockSpec((_G, _TOPK, 1), lambda b: (b, 0, 0)),
            pl.BlockSpec((_G, _TOPK, 1), lambda b: (b, 0, 0)),
            pl.BlockSpec((_G, _TOPK, 4), lambda b: (b, 0, 0)),
        ],
        out_shape=[
            jax.ShapeDtypeStruct((B, _TOPK, 1), jnp.float32),
            jax.ShapeDtypeStruct((B, _TOPK, 1), jnp.int32),
            jax.ShapeDtypeStruct((B, _TOPK, 4), jnp.int32),
        ],
        scratch_shapes=[pltpu.VMEM((_G, _NCHUNK, _CHUNK), jnp.float32)],
    )(pa, pb, pred_boxes, target_sizes.reshape(B, 1, 2))
    return scores[..., 0], labels[..., 0], boxes


# speedup vs baseline: 12.3692x; 1.6381x over previous
import jax
import jax.numpy as jnp
from jax.experimental import pallas as pl
from jax.experimental.pallas import tpu as pltpu

_TOPK = 100
_C = 91
_NCHUNK = 1024
_CHUNK = 80
_G = 16


def _topk_body(pa_ref, pb_ref, boxes_ref, ts_ref, scores_ref, labels_ref,
               boxes_out_ref):
    jio = jax.lax.broadcasted_iota(jnp.int32, (_CHUNK, _NCHUNK), 0)
    cio = jax.lax.broadcasted_iota(jnp.int32, (1, _NCHUNK), 1)
    lio = jax.lax.broadcasted_iota(jnp.int32, (1, _CHUNK), 1)

    cm_rows, ai_rows = [], []
    for b in range(_G):
        pa = pa_ref[b]
        cm0 = jnp.max(pa, axis=0, keepdims=True)
        ai0 = jnp.min(jnp.where(pa == cm0, jio, _CHUNK), axis=0,
                      keepdims=True)
        cm_rows.append(cm0)
        ai_rows.append(ai0)
    cm_all = jnp.concatenate(cm_rows, axis=0)
    ai_all = jnp.concatenate(ai_rows, axis=0)

    tsf = ts_ref[...].reshape(_G, 2).astype(jnp.float32)
    wh = jnp.concatenate([tsf[:, 1:2], tsf[:, 0:1]], axis=1)
    scale = jnp.concatenate([wh, wh], axis=1)

    def step(i, carry):
        cm_all, ai_all = carry
        m_col = jnp.max(cm_all, axis=1, keepdims=True)
        ci_col = jnp.min(jnp.where(cm_all == m_col, cio, _NCHUNK),
                         axis=1, keepdims=True)
        j_col = jnp.min(jnp.where(cio == ci_col, ai_all, _CHUNK),
                        axis=1, keepdims=True)
        flat = ci_col * _CHUNK + j_col
        q_col = flat // _C
        cls_col = flat % _C
        scores_ref[:, pl.ds(i, 1), :] = m_col.reshape(_G, 1, 1)
        labels_ref[:, pl.ds(i, 1), :] = cls_col.reshape(_G, 1, 1)

        box_rows, work_rows = [], []
        for b in range(_G):
            box_rows.append(boxes_ref[b, pl.ds(q_col[b, 0], 1), :])
            work_rows.append(pb_ref[b, pl.ds(ci_col[b, 0], 1), :])
        bx = jnp.concatenate(box_rows, axis=0)
        rows = jnp.concatenate(work_rows, axis=0)

        cxy = bx[:, 0:2]
        bwh = bx[:, 2:4]
        xyxy = jnp.concatenate([cxy - 0.5 * bwh, cxy + 0.5 * bwh], axis=1)
        xyxy = (xyxy * scale).astype(jnp.int32)

        rows = jnp.where(lio == j_col, -1.0, rows)
        nm_col = jnp.max(rows, axis=1, keepdims=True)
        na_col = jnp.min(jnp.where(rows == nm_col, lio, _CHUNK),
                         axis=1, keepdims=True)
        for b in range(_G):
            boxes_out_ref[b, pl.ds(i, 1), :] = xyxy[b:b + 1]
            pb_ref[b, pl.ds(ci_col[b, 0], 1), :] = rows[b:b + 1]

        cm_all = jnp.where(cio == ci_col, nm_col, cm_all)
        ai_all = jnp.where(cio == ci_col, na_col, ai_all)
        return cm_all, ai_all

    jax.lax.fori_loop(0, _TOPK, step, (cm_all, ai_all))


@jax.jit
def kernel(pred_logits, pred_boxes, target_sizes):
    B, Q, C = pred_logits.shape
    prob = jax.nn.sigmoid(pred_logits).reshape(B, Q * C)
    flat = jnp.pad(prob, ((0, 0), (0, _NCHUNK * _CHUNK - Q * C)),
                   constant_values=-1.0)
    pb = flat.reshape(B, _NCHUNK, _CHUNK)
    pa = pb.transpose(0, 2, 1)
    scores, labels, boxes = pl.pallas_call(
        _topk_body,
        grid=(B // _G,),
        in_specs=[
            pl.BlockSpec((_G, _CHUNK, _NCHUNK), lambda b: (b, 0, 0)),
            pl.BlockSpec((_G, _NCHUNK, _CHUNK), lambda b: (b, 0, 0)),
            pl.BlockSpec((_G, Q, 4), lambda b: (b, 0, 0)),
            pl.BlockSpec((_G, 1, 2), lambda b: (b, 0, 0)),
        ],
        out_specs=[
            pl.BlockSpec((_G, _TOPK, 1), lambda b: (b, 0, 0)),
            pl.BlockSpec((_G, _TOPK, 1), lambda b: (b, 0, 0)),
            pl.BlockSpec((_G, _TOPK, 4), lambda b: (b, 0, 0)),
        ],
        out_shape=[
            jax.ShapeDtypeStruct((B, _TOPK, 1), jnp.float32),
            jax.ShapeDtypeStruct((B, _TOPK, 1), jnp.int32),
            jax.ShapeDtypeStruct((B, _TOPK, 4), jnp.int32),
        ],
    )(pa, pb, pred_boxes, target_sizes.reshape(B, 1, 2))
    return scores[..., 0], labels[..., 0], boxes
